# baseline (speedup 1.0000x reference)
import jax
import jax.numpy as jnp
from jax.experimental import pallas as pl
from jax.experimental.pallas import tpu as pltpu

_B, _S, _D = 16, 2048, 1024
_S_BLK = 1024
_NJ = _S // _S_BLK


def _body(lens_ref, mask_ref, x_ref, o_ref):
    del lens_ref
    j = pl.program_id(1)
    m_row = mask_ref[0, 0, :]
    length = jnp.sum(m_row)
    m_blk = mask_ref[0, 0, pl.ds(j * _S_BLK, _S_BLK)]
    pos = jax.lax.broadcasted_iota(jnp.int32, (_S_BLK, 1), 0) + j * _S_BLK
    keep = (m_blk.reshape(_S_BLK, 1) != 0) & (pos < length)
    o_ref[0] = x_ref[0] * keep.astype(jnp.float32)


def _x_index(b, j, lens_ref):
    j_eff = jnp.where(lens_ref[b] > j * _S_BLK, j, jnp.maximum(j - 1, 0))
    return (b, j_eff, 0)


def kernel(x, mask):
    m3 = mask.astype(jnp.int32).reshape(_B, 1, _S)
    lens = jnp.sum(m3[:, 0, :], axis=1).astype(jnp.int32)
    grid_spec = pltpu.PrefetchScalarGridSpec(
        num_scalar_prefetch=1,
        grid=(_B, _NJ),
        in_specs=[
            pl.BlockSpec((1, 1, _S), lambda b, j, lens_ref: (b, 0, 0)),
            pl.BlockSpec((1, _S_BLK, _D), _x_index),
        ],
        out_specs=pl.BlockSpec((1, _S_BLK, _D), lambda b, j, lens_ref: (b, j, 0)),
    )
    return pl.pallas_call(
        _body,
        grid_spec=grid_spec,
        out_shape=jax.ShapeDtypeStruct((_B, _S, _D), jnp.float32),
    )(lens, m3, x)
